# baseline (speedup 1.0000x reference)
import functools

import jax
import jax.numpy as jnp
from jax import lax
from jax.experimental import pallas as pl
from jax.experimental.pallas import tpu as pltpu
from jax.experimental.pallas import tpu_sc as plsc

NC = 2
NS = 16
NW = NC * NS

CHUNK = 160
PD = 128


def _gather_call(flat_idx, table_pairs, dim):
    b = flat_idx.shape[0]
    b_per_w = b // NW
    n_chunks = b_per_w // CHUNK
    n_pairs = n_chunks // 2
    mesh = plsc.VectorSubcoreMesh(core_axis_name="c", subcore_axis_name="s")

    @functools.partial(
        pl.kernel,
        mesh=mesh,
        out_type=jax.ShapeDtypeStruct((b, dim), jnp.float32),
        scratch_types=[
            pltpu.VMEM((CHUNK,), jnp.int32),
            pltpu.VMEM((CHUNK,), jnp.int32),
            pltpu.VMEM((CHUNK,), jnp.int32),
            pltpu.VMEM((CHUNK,), jnp.int32),
            pltpu.VMEM((CHUNK, PD), jnp.float32),
            pltpu.VMEM((CHUNK, PD), jnp.float32),
            pltpu.VMEM((CHUNK, dim), jnp.float32),
            pltpu.VMEM((CHUNK, dim), jnp.float32),
            pltpu.SemaphoreType.DMA((2,)),
            pltpu.SemaphoreType.DMA((2,)),
        ],
        compiler_params=pltpu.CompilerParams(use_tc_tiling_on_sc=True),
    )
    def run(idx_hbm, table_hbm, out_hbm, idx_a, idx_b, idx2_a, idx2_b,
            rows_a, rows_b, sel_a, sel_b, gsem, osem):
        idx_s = (idx_a, idx_b)
        idx2_s = (idx2_a, idx2_b)
        rows_s = (rows_a, rows_b)
        sel_s = (sel_a, sel_b)
        wid = lax.axis_index("s") * NC + lax.axis_index("c")
        base = pl.multiple_of(wid * b_per_w, b_per_w)

        def prep_gather(g, slot):
            off = pl.multiple_of(base + g * CHUNK, CHUNK)
            pltpu.sync_copy(idx_hbm.at[pl.ds(off, CHUNK)], idx_s[slot])

            def halve(k, carry):
                sl = pl.ds(k * 16, 16)
                idx2_s[slot][sl] = lax.shift_right_logical(idx_s[slot][sl], 1)
                return carry

            lax.fori_loop(0, CHUNK // 16, halve, 0)
            pltpu.async_copy(table_hbm.at[idx2_s[slot]], rows_s[slot],
                             gsem.at[slot])

        def select_half(slot):
            def grp(g16, carry):
                par = (idx_s[slot][pl.ds(g16 * 16, 16)] & 1) * dim
                for j in range(16):
                    off = par[j]
                    r = g16 * 16 + j
                    for cb in range(dim // 16):
                        sel_s[slot][r, pl.ds(cb * 16, 16)] = (
                            rows_s[slot][r, pl.ds(off + cb * 16, 16)])
                return carry

            lax.fori_loop(0, CHUNK // 16, grp, 0)

        def start_write(g, slot):
            off = pl.multiple_of(base + g * CHUNK, CHUNK)
            pltpu.async_copy(sel_s[slot], out_hbm.at[pl.ds(off, CHUNK)],
                             osem.at[slot])

        def wait_gather(slot):
            pltpu.make_async_copy(table_hbm.at[pl.ds(0, CHUNK)],
                                  rows_s[slot], gsem.at[slot]).wait()

        def wait_write(slot):
            pltpu.make_async_copy(sel_s[slot],
                                  out_hbm.at[pl.ds(0, CHUNK)],
                                  osem.at[slot]).wait()

        prep_gather(0, 0)
        prep_gather(1, 1)

        def pair(p, carry):
            g0 = p * 2
            for (goff, slot) in ((0, 0), (1, 1)):
                g = g0 + goff
                wait_gather(slot)

                @pl.when(p > 0)
                def _():
                    wait_write(slot)

                select_half(slot)
                start_write(g, slot)

                @pl.when(g + 2 < n_chunks)
                def _():
                    prep_gather(g + 2, slot)
            return carry

        lax.fori_loop(0, n_pairs, pair, 0)
        wait_write(0)
        wait_write(1)

    return run(flat_idx, table_pairs)


def kernel(indices, table):
    nb, ns = indices.shape
    dim = table.shape[1]
    flat_idx = indices.T.reshape(-1).astype(jnp.int32)
    table_pairs = table.reshape(table.shape[0] // 2, 2 * dim)
    out_f = _gather_call(flat_idx, table_pairs, dim)
    return out_f.reshape(ns, nb, dim).transpose(1, 0, 2)


# speedup vs baseline: 1.2466x; 1.2466x over previous
import functools

import jax
import jax.numpy as jnp
from jax import lax
from jax.experimental import pallas as pl
from jax.experimental.pallas import tpu as pltpu
from jax.experimental.pallas import tpu_sc as plsc

NC = 2
NS = 16
NW = NC * NS

CHUNK = 400
PD = 128


def _gather_call(flat_idx, table_p, dim):
    b = flat_idx.shape[0]
    b_per_w = b // NW
    n_chunks = b_per_w // CHUNK
    n_pairs = n_chunks // 2
    mesh = plsc.VectorSubcoreMesh(core_axis_name="c", subcore_axis_name="s")

    @functools.partial(
        pl.kernel,
        mesh=mesh,
        out_type=jax.ShapeDtypeStruct((b, PD), jnp.float32),
        scratch_types=[
            pltpu.VMEM((b_per_w,), jnp.int32),
            pltpu.VMEM((CHUNK, PD), jnp.float32),
            pltpu.VMEM((CHUNK, PD), jnp.float32),
            pltpu.SemaphoreType.DMA((2,)),
            pltpu.SemaphoreType.DMA((2,)),
        ],
        compiler_params=pltpu.CompilerParams(use_tc_tiling_on_sc=True),
    )
    def run(idx_hbm, table_hbm, out_hbm, idx_v, rows_a, rows_b, gsem, osem):
        rows_s = (rows_a, rows_b)
        wid = lax.axis_index("s") * NC + lax.axis_index("c")
        base = pl.multiple_of(wid * b_per_w, b_per_w)
        pltpu.sync_copy(idx_hbm.at[pl.ds(base, b_per_w)], idx_v)

        def start_gather(g, slot):
            pltpu.async_copy(
                table_hbm.at[idx_v.at[pl.ds(g * CHUNK, CHUNK)]],
                rows_s[slot],
                gsem.at[slot],
            )

        def start_write(g, slot):
            off = pl.multiple_of(base + g * CHUNK, CHUNK)
            pltpu.async_copy(rows_s[slot], out_hbm.at[pl.ds(off, CHUNK)],
                             osem.at[slot])

        def wait_gather(slot):
            pltpu.make_async_copy(table_hbm.at[pl.ds(0, CHUNK)],
                                  rows_s[slot], gsem.at[slot]).wait()

        def wait_write(slot):
            pltpu.make_async_copy(rows_s[slot],
                                  out_hbm.at[pl.ds(0, CHUNK)],
                                  osem.at[slot]).wait()

        start_gather(0, 0)

        def pair(p, carry):
            g0 = p * 2
            wait_gather(0)

            @pl.when(p > 0)
            def _():
                wait_write(1)

            start_gather(g0 + 1, 1)
            start_write(g0, 0)
            wait_gather(1)

            @pl.when(p + 1 < n_pairs)
            def _():
                wait_write(0)
                start_gather(g0 + 2, 0)

            start_write(g0 + 1, 1)
            return carry

        lax.fori_loop(0, n_pairs, pair, 0)
        wait_write(0)
        wait_write(1)

    return run(flat_idx, table_p)


def kernel(indices, table):
    nb, ns = indices.shape
    dim = table.shape[1]
    flat_idx = indices.T.reshape(-1).astype(jnp.int32)
    table_p = jnp.pad(table, ((0, 0), (0, PD - dim)))
    out_p = _gather_call(flat_idx, table_p, dim)
    return out_p.reshape(ns, nb, PD)[:, :, :dim].transpose(1, 0, 2)
